# baseline (speedup 1.0000x reference)
import functools

import jax
import jax.numpy as jnp
from jax import lax
from jax.experimental import pallas as pl
from jax.experimental.pallas import tpu as pltpu
from jax.experimental.pallas import tpu_sc as plsc

D = 64
CHU = 64
F = 50
FP = 56
UPC = 4
CROWS = UPC * F
SUBS = (104, 96)
NBUF = 7
K = 5


def _mesh():
    return plsc.VectorSubcoreMesh(core_axis_name="c", subcore_axis_name="s")


@functools.lru_cache(maxsize=None)
def _make_users(B):
    info = plsc.get_sparse_core_info()
    NC, NS = info.num_cores, info.num_subcores
    NW = NC * NS
    rows_u = B // NW

    @functools.partial(
        pl.kernel,
        mesh=_mesh(),
        out_type=jax.ShapeDtypeStruct((B, 2 * D), jnp.float32),
        scratch_types=[
            pltpu.VMEM((rows_u,), jnp.int32),
            pltpu.VMEM((CHU * 2, 2 * D), jnp.float32),
            pltpu.SemaphoreType.DMA,
            pltpu.SemaphoreType.DMA,
        ],
        compiler_params=pltpu.CompilerParams(use_tc_tiling_on_sc=False),
    )
    def k(users_hbm, utab, uout, uidx, urows, gsem, ssem):
        wid = lax.axis_index("s") * NC + lax.axis_index("c")
        nb = rows_u // CHU

        def ubuf(g):
            return urows.at[pl.ds((g % 2) * CHU, CHU)]

        def u_fire(g):
            pltpu.async_copy(
                utab.at[uidx.at[pl.ds(g * CHU, CHU)]], ubuf(g), gsem)

        pltpu.sync_copy(users_hbm.at[pl.ds(wid * rows_u, rows_u)], uidx)
        u_fire(0)
        u_fire(1)
        for g in range(nb):
            pltpu.make_async_copy(utab.at[pl.ds(0, CHU)], ubuf(0),
                                  gsem).wait()
            ob = pl.multiple_of(wid * rows_u + g * CHU, 8)
            pltpu.async_copy(ubuf(g).at[:, pl.ds(0, D)],
                             uout.at[pl.ds(ob, CHU), pl.ds(0, D)], ssem)
            pltpu.make_async_copy(
                uout.at[pl.ds(wid * rows_u, CHU), pl.ds(0, D)],
                urows.at[pl.ds(0, CHU), pl.ds(0, D)], ssem).wait()
            if g + 2 < nb:
                u_fire(g + 2)

    return k


@functools.lru_cache(maxsize=None)
def _make_items(B):
    info = plsc.get_sparse_core_info()
    NC, NS = info.num_cores, info.num_subcores
    NW = NC * NS
    users_w = B // NW
    rows_f = users_w * F
    n_chunks = users_w // UPC

    @functools.partial(
        pl.kernel,
        mesh=_mesh(),
        out_type=jax.ShapeDtypeStruct((B * FP, 2 * D), jnp.float32),
        scratch_types=[
            pltpu.VMEM((rows_f,), jnp.int32),
            pltpu.VMEM((NBUF * CROWS, D), jnp.float32),
            pltpu.SemaphoreType.DMA,
            pltpu.SemaphoreType.DMA,
        ],
        compiler_params=pltpu.CompilerParams(use_tc_tiling_on_sc=False),
    )
    def k(feats_hbm, itab, fout, fidx, rows, gsem, ssem):
        wid = lax.axis_index("s") * NC + lax.axis_index("c")
        base_u = wid * users_w

        def fire_gather(g):
            b = (g % NBUF) * CROWS
            pltpu.async_copy(
                itab.at[fidx.at[pl.ds(g * CROWS, SUBS[0])]],
                rows.at[pl.ds(b, SUBS[0])], gsem)
            pltpu.async_copy(
                itab.at[fidx.at[pl.ds(g * CROWS + SUBS[0], SUBS[1])]],
                rows.at[pl.ds(b + SUBS[0], SUBS[1])], gsem)

        def wait_gather():
            pltpu.make_async_copy(itab.at[pl.ds(0, SUBS[0])],
                                  rows.at[pl.ds(0, SUBS[0])], gsem).wait()
            pltpu.make_async_copy(itab.at[pl.ds(0, SUBS[1])],
                                  rows.at[pl.ds(0, SUBS[1])], gsem).wait()

        def fire_store(g):
            b = (g % NBUF) * CROWS
            for u in range(UPC):
                uu = base_u + g * UPC + u
                ob = pl.multiple_of(uu * FP, 8)
                pltpu.async_copy(
                    rows.at[pl.ds(b + u * F, F)],
                    fout.at[pl.ds(ob, F), pl.ds(0, D)], ssem)

        def drain_store():
            for _ in range(UPC):
                pltpu.make_async_copy(
                    fout.at[pl.ds(base_u * FP, F), pl.ds(0, D)],
                    rows.at[pl.ds(0, F)], ssem).wait()

        pltpu.sync_copy(feats_hbm.at[pl.ds(wid * rows_f, rows_f)], fidx)

        for g in range(K):
            fire_gather(g)
        for g in range(2):
            wait_gather()
            fire_store(g)
            fire_gather(g + K)

        def body(g, _):
            wait_gather()
            fire_store(g)
            drain_store()
            fire_gather(g + K)
            return 0

        lax.fori_loop(2, n_chunks - K, body, 0)

        for g in range(n_chunks - K, n_chunks):
            wait_gather()
            fire_store(g)
            drain_store()
        drain_store()
        drain_store()

    return k


def kernel(users, feats, user_table, item_table):
    B = users.shape[0]
    fout = _make_items(B)(feats.reshape(-1), item_table)
    utab_p = -jnp.pad(-user_table, ((0, 0), (0, D)))
    uout = _make_users(B)(users, utab_p)
    return (uout[:, :D],
            fout.reshape(B, FP, 2 * D)[:, :F, :D])


# speedup vs baseline: 1.1666x; 1.1666x over previous
import functools

import jax
import jax.numpy as jnp
from jax import lax
from jax.experimental import pallas as pl
from jax.experimental.pallas import tpu as pltpu
from jax.experimental.pallas import tpu_sc as plsc

D = 64
CHU = 64
F = 50
FP = 56
UPC = 4
CROWS = UPC * F
SUBS = (104, 96)
NBUF = 7
K = 5


def _mesh():
    return plsc.VectorSubcoreMesh(core_axis_name="c", subcore_axis_name="s")


@functools.lru_cache(maxsize=None)
def _make_users(B):
    info = plsc.get_sparse_core_info()
    NC, NS = info.num_cores, info.num_subcores
    NW = NC * NS
    rows_u = B // NW

    @functools.partial(
        pl.kernel,
        mesh=_mesh(),
        out_type=jax.ShapeDtypeStruct((B, 2 * D), jnp.float32),
        scratch_types=[
            pltpu.VMEM((rows_u,), jnp.int32),
            pltpu.VMEM((CHU * 2, 2 * D), jnp.float32),
            pltpu.SemaphoreType.DMA,
            pltpu.SemaphoreType.DMA,
        ],
        compiler_params=pltpu.CompilerParams(use_tc_tiling_on_sc=False),
    )
    def k(users_hbm, utab, uout, uidx, urows, gsem, ssem):
        wid = lax.axis_index("s") * NC + lax.axis_index("c")
        nb = rows_u // CHU

        def ubuf(g):
            return urows.at[pl.ds((g % 2) * CHU, CHU)]

        def u_fire(g):
            pltpu.async_copy(
                utab.at[uidx.at[pl.ds(g * CHU, CHU)]], ubuf(g), gsem)

        pltpu.sync_copy(users_hbm.at[pl.ds(wid * rows_u, rows_u)], uidx)
        u_fire(0)
        u_fire(1)
        for g in range(nb):
            pltpu.make_async_copy(utab.at[pl.ds(0, CHU)], ubuf(0),
                                  gsem).wait()
            ob = pl.multiple_of(wid * rows_u + g * CHU, 8)
            pltpu.async_copy(ubuf(g).at[:, pl.ds(0, D)],
                             uout.at[pl.ds(ob, CHU), pl.ds(0, D)], ssem)
            pltpu.make_async_copy(
                uout.at[pl.ds(wid * rows_u, CHU), pl.ds(0, D)],
                urows.at[pl.ds(0, CHU), pl.ds(0, D)], ssem).wait()
            if g + 2 < nb:
                u_fire(g + 2)

    return k


@functools.lru_cache(maxsize=None)
def _make_items(B):
    info = plsc.get_sparse_core_info()
    NC, NS = info.num_cores, info.num_subcores
    NW = NC * NS
    users_w = B // NW
    rows_f = users_w * F
    n_chunks = users_w // UPC

    @functools.partial(
        pl.kernel,
        mesh=_mesh(),
        out_type=jax.ShapeDtypeStruct((B * FP, 2 * D), jnp.float32),
        scratch_types=[
            pltpu.VMEM((rows_f,), jnp.int32),
            pltpu.VMEM((NBUF * CROWS, D), jnp.float32),
            pltpu.SemaphoreType.DMA,
            pltpu.SemaphoreType.DMA,
        ],
        compiler_params=pltpu.CompilerParams(use_tc_tiling_on_sc=False),
    )
    def k(feats_hbm, itab, fout, fidx, rows, gsem, ssem):
        wid = lax.axis_index("s") * NC + lax.axis_index("c")
        base_u = wid * users_w

        def fire_gather(g):
            b = (g % NBUF) * CROWS
            pltpu.async_copy(
                itab.at[fidx.at[pl.ds(g * CROWS, SUBS[0])]],
                rows.at[pl.ds(b, SUBS[0])], gsem)
            pltpu.async_copy(
                itab.at[fidx.at[pl.ds(g * CROWS + SUBS[0], SUBS[1])]],
                rows.at[pl.ds(b + SUBS[0], SUBS[1])], gsem)

        def wait_gather():
            pltpu.make_async_copy(itab.at[pl.ds(0, SUBS[0])],
                                  rows.at[pl.ds(0, SUBS[0])], gsem).wait()
            pltpu.make_async_copy(itab.at[pl.ds(0, SUBS[1])],
                                  rows.at[pl.ds(0, SUBS[1])], gsem).wait()

        def fire_store(g):
            b = (g % NBUF) * CROWS
            for u in range(UPC):
                uu = base_u + g * UPC + u
                ob = pl.multiple_of(uu * FP, 8)
                pltpu.async_copy(
                    rows.at[pl.ds(b + u * F, F)],
                    fout.at[pl.ds(ob, F), pl.ds(0, D)], ssem)

        def drain_store():
            for _ in range(UPC):
                pltpu.make_async_copy(
                    fout.at[pl.ds(base_u * FP, F), pl.ds(0, D)],
                    rows.at[pl.ds(0, F)], ssem).wait()

        pltpu.sync_copy(feats_hbm.at[pl.ds(wid * rows_f, rows_f)], fidx)

        for g in range(K):
            fire_gather(g)
        for g in range(2):
            wait_gather()
            fire_store(g)
            fire_gather(g + K)

        def body(g, _):
            wait_gather()
            fire_store(g)
            drain_store()
            fire_gather(g + K)
            return 0

        lax.fori_loop(2, n_chunks - K, body, 0)

        for g in range(n_chunks - K, n_chunks):
            wait_gather()
            fire_store(g)
            drain_store()
        drain_store()
        drain_store()

    return k


def kernel(users, feats, user_table, item_table):
    B = users.shape[0]
    fout = _make_items(B)(feats.reshape(-1), item_table)
    utab_p = jnp.pad(user_table, ((0, 0), (0, D)))
    uout = _make_users(B)(users, utab_p)
    return (uout[:, :D],
            fout.reshape(B, FP, 2 * D)[:, :F, :D])
